# baseline (speedup 1.0000x reference)
import functools

import jax
import jax.numpy as jnp
from jax import lax
from jax.experimental import pallas as pl
from jax.experimental.pallas import tpu as pltpu
from jax.experimental.pallas import tpu_sc as plsc

_NC, _NS = 2, 16
_NW = _NC * _NS
_CHUNK = 128


def _dist_argmax_body(x_ref, et_ref, dist_ref, idx_ref):
    k = dist_ref.shape[1]
    nchunks = 8
    ck = k // nchunks
    x = x_ref[...]
    if True:
        for c in range(nchunks):
            d = lax.dot_general(
                x, et_ref[c * ck:(c + 1) * ck, :],
                dimension_numbers=(((1,), (1,)), ((), ())),
                preferred_element_type=jnp.float32,
            )
            dist_ref[:, c * ck:(c + 1) * ck] = d
        idx_ref[0, 0, :] = jnp.zeros((x.shape[0],), jnp.int32)
        return
    chunk_max = []
    chunk_arg = []
    for c in range(nchunks):
        d = lax.dot_general(
            x, et_ref[c * ck:(c + 1) * ck, :],
            dimension_numbers=(((1,), (1,)), ((), ())),
            preferred_element_type=jnp.float32,
        )
        dist_ref[:, c * ck:(c + 1) * ck] = d
        m = jnp.max(d, axis=1, keepdims=True)
        iota = lax.broadcasted_iota(jnp.int32, d.shape, 1).astype(jnp.float32)
        arg = jnp.min(jnp.where(d == m, iota, float(ck)), axis=1)
        chunk_max.append(m[:, 0])
        chunk_arg.append(arg + float(c * ck))
    vals = jnp.stack(chunk_max, axis=1)
    args = jnp.stack(chunk_arg, axis=1)
    gm = jnp.max(vals, axis=1, keepdims=True)
    pick = jnp.min(jnp.where(vals == gm, args, float(k)), axis=1)
    idx_ref[0, 0, :] = pick.astype(jnp.int32)


def _dist_argmax(xf, et, row_tile):
    bn, d = xf.shape
    k = et.shape[0]
    nt = bn // row_tile
    dist, idx3 = pl.pallas_call(
        _dist_argmax_body,
        grid=(nt,),
        in_specs=[
            pl.BlockSpec((row_tile, d), lambda i: (i, 0)),
            pl.BlockSpec((k, d), lambda i: (0, 0)),
        ],
        out_specs=[
            pl.BlockSpec((row_tile, k), lambda i: (i, 0)),
            pl.BlockSpec((1, 1, row_tile), lambda i: (i, 0, 0)),
        ],
        out_shape=[
            jax.ShapeDtypeStruct((bn, k), jnp.float32),
            jax.ShapeDtypeStruct((nt, 1, row_tile), jnp.int32),
        ],
        compiler_params=pltpu.CompilerParams(
            dimension_semantics=("arbitrary",),
        ),
    )(xf, et)
    return dist, idx3.reshape(bn)


def _sc_gather_body(table_hbm, idx_hbm, out_hbm, idx_v, rows_v, sem):
    wid = lax.axis_index("s") * _NC + lax.axis_index("c")
    pltpu.sync_copy(idx_hbm.at[wid], idx_v)
    nch = idx_v.shape[0]
    for j in range(nch):
        pltpu.async_copy(table_hbm.at[idx_v.at[j]], rows_v.at[j], sem).wait()
    pltpu.sync_copy(rows_v, out_hbm.at[wid])


def _sc_gather(table, idx):
    k, d = table.shape
    bn = idx.shape[0]
    b_per_w = bn // _NW
    nch = b_per_w // _CHUNK
    idx3 = idx.reshape(_NW, nch, _CHUNK)
    gathered = pl.kernel(
        _sc_gather_body,
        mesh=plsc.VectorSubcoreMesh(core_axis_name="c", subcore_axis_name="s"),
        out_type=jax.ShapeDtypeStruct((_NW, nch, _CHUNK, d), jnp.float32),
        scratch_types=[
            pltpu.VMEM((nch, _CHUNK), jnp.int32),
            pltpu.VMEM((nch, _CHUNK, d), jnp.float32),
            pltpu.SemaphoreType.DMA,
        ],
    )(table, idx3)
    return gathered.reshape(bn, d)


def kernel(x, embed):
    x = x.astype(jnp.float32)
    b, n, d = x.shape
    k = embed.shape[1]
    xf = x.reshape(b * n, d)
    table = embed[0].astype(jnp.float32)

    dist, idx_flat = _dist_argmax(xf, table, row_tile=512)

    quantize = _sc_gather(table, idx_flat).reshape(b, n, d)
    embed_ind = idx_flat.reshape(b, n)
    dist_out = dist.reshape(1, b, n, k)
    return quantize, embed_ind, dist_out


# speedup vs baseline: 3.7482x; 3.7482x over previous
import functools

import jax
import jax.numpy as jnp
from jax import lax
from jax.experimental import pallas as pl
from jax.experimental.pallas import tpu as pltpu
from jax.experimental.pallas import tpu_sc as plsc

_NC, _NS = 2, 16
_NW = _NC * _NS
_CHUNK = 128


def _dist_argmax_body(x_ref, et_ref, dist_ref, idx_ref):
    k = dist_ref.shape[1]
    nchunks = 8
    ck = k // nchunks
    x = x_ref[...]
    chunk_max = []
    chunk_arg = []
    for c in range(nchunks):
        d = lax.dot_general(
            x, et_ref[c * ck:(c + 1) * ck, :],
            dimension_numbers=(((1,), (1,)), ((), ())),
            preferred_element_type=jnp.float32,
        )
        dist_ref[:, c * ck:(c + 1) * ck] = d
        m = jnp.max(d, axis=1, keepdims=True)
        iota = lax.broadcasted_iota(jnp.int32, d.shape, 1).astype(jnp.float32)
        arg = jnp.min(jnp.where(d == m, iota, float(ck)), axis=1)
        chunk_max.append(m[:, 0])
        chunk_arg.append(arg + float(c * ck))
    vals = jnp.stack(chunk_max, axis=1)
    args = jnp.stack(chunk_arg, axis=1)
    gm = jnp.max(vals, axis=1, keepdims=True)
    pick = jnp.min(jnp.where(vals == gm, args, float(k)), axis=1)
    idx_ref[0, 0, :] = pick.astype(jnp.int32)


def _dist_argmax(xf, et, row_tile):
    bn, d = xf.shape
    k = et.shape[0]
    nt = bn // row_tile
    dist, idx3 = pl.pallas_call(
        _dist_argmax_body,
        grid=(nt,),
        in_specs=[
            pl.BlockSpec((row_tile, d), lambda i: (i, 0)),
            pl.BlockSpec((k, d), lambda i: (0, 0)),
        ],
        out_specs=[
            pl.BlockSpec((row_tile, k), lambda i: (i, 0)),
            pl.BlockSpec((1, 1, row_tile), lambda i: (i, 0, 0)),
        ],
        out_shape=[
            jax.ShapeDtypeStruct((bn, k), jnp.float32),
            jax.ShapeDtypeStruct((nt, 1, row_tile), jnp.int32),
        ],
        compiler_params=pltpu.CompilerParams(
            dimension_semantics=("parallel",),
        ),
    )(xf, et)
    return dist, idx3.reshape(bn)


def _sc_gather_body(table_hbm, idx_hbm, out_hbm, idx_v, rows_v, sem):
    wid = lax.axis_index("s") * _NC + lax.axis_index("c")
    pltpu.sync_copy(idx_hbm.at[wid], idx_v)
    nch = idx_v.shape[0]
    for j in range(nch):
        pltpu.async_copy(table_hbm.at[idx_v.at[j]], rows_v.at[j], sem).wait()
    pltpu.sync_copy(rows_v, out_hbm.at[wid])


def _sc_gather(table, idx):
    k, d = table.shape
    bn = idx.shape[0]
    b_per_w = bn // _NW
    nch = b_per_w // _CHUNK
    idx3 = idx.reshape(_NW, nch, _CHUNK)
    gathered = pl.kernel(
        _sc_gather_body,
        mesh=plsc.VectorSubcoreMesh(core_axis_name="c", subcore_axis_name="s"),
        out_type=jax.ShapeDtypeStruct((_NW, nch, _CHUNK, d), jnp.float32),
        scratch_types=[
            pltpu.VMEM((nch, _CHUNK), jnp.int32),
            pltpu.VMEM((nch, _CHUNK, d), jnp.float32),
            pltpu.SemaphoreType.DMA,
        ],
    )(table, idx3)
    return gathered.reshape(bn, d)


def kernel(x, embed):
    x = x.astype(jnp.float32)
    b, n, d = x.shape
    k = embed.shape[1]
    xf = x.reshape(b * n, d)
    table = embed[0].astype(jnp.float32)

    dist, idx_flat = _dist_argmax(xf, table, row_tile=512)

    quantize = _sc_gather(table, idx_flat).reshape(b, n, d)
    embed_ind = idx_flat.reshape(b, n)
    dist_out = dist.reshape(1, b, n, k)
    return quantize, embed_ind, dist_out


# speedup vs baseline: 3.7647x; 1.0044x over previous
import functools

import jax
import jax.numpy as jnp
from jax import lax
from jax.experimental import pallas as pl
from jax.experimental.pallas import tpu as pltpu
from jax.experimental.pallas import tpu_sc as plsc

_NC, _NS = 2, 16
_NW = _NC * _NS
_CHUNK = 128


def _dist_argmax_body(x_ref, et_ref, dist_ref, idx_ref):
    k = dist_ref.shape[1]
    nchunks = 8
    ck = k // nchunks
    x = x_ref[...]
    chunk_max = []
    chunk_arg = []
    for c in range(nchunks):
        d = lax.dot_general(
            x, et_ref[c * ck:(c + 1) * ck, :],
            dimension_numbers=(((1,), (1,)), ((), ())),
            preferred_element_type=jnp.float32,
        )
        dist_ref[:, c * ck:(c + 1) * ck] = d
        m = jnp.max(d, axis=1, keepdims=True)
        iota = lax.broadcasted_iota(jnp.int32, d.shape, 1).astype(jnp.float32)
        arg = jnp.min(jnp.where(d == m, iota, float(ck)), axis=1)
        chunk_max.append(m[:, 0])
        chunk_arg.append(arg + float(c * ck))
    vals = jnp.stack(chunk_max, axis=1)
    args = jnp.stack(chunk_arg, axis=1)
    gm = jnp.max(vals, axis=1, keepdims=True)
    pick = jnp.min(jnp.where(vals == gm, args, float(k)), axis=1)
    idx_ref[0, 0, :] = pick.astype(jnp.int32)


def _dist_argmax(xf, et, row_tile):
    bn, d = xf.shape
    k = et.shape[0]
    nt = bn // row_tile
    dist, idx3 = pl.pallas_call(
        _dist_argmax_body,
        grid=(nt,),
        in_specs=[
            pl.BlockSpec((row_tile, d), lambda i: (i, 0)),
            pl.BlockSpec((k, d), lambda i: (0, 0)),
        ],
        out_specs=[
            pl.BlockSpec((row_tile, k), lambda i: (i, 0)),
            pl.BlockSpec((1, 1, row_tile), lambda i: (i, 0, 0)),
        ],
        out_shape=[
            jax.ShapeDtypeStruct((bn, k), jnp.float32),
            jax.ShapeDtypeStruct((nt, 1, row_tile), jnp.int32),
        ],
        compiler_params=pltpu.CompilerParams(
            dimension_semantics=("parallel",),
        ),
    )(xf, et)
    return dist, idx3.reshape(bn)


def _sc_gather_body(table_hbm, idx_hbm, out_hbm, idx_v, rows_v, *sems):
    wid = lax.axis_index("s") * _NC + lax.axis_index("c")
    pltpu.sync_copy(idx_hbm.at[wid], idx_v)
    nch = idx_v.shape[0]
    osem = sems[nch]
    gathers = [
        pltpu.async_copy(table_hbm.at[idx_v.at[j]], rows_v.at[j], sems[j])
        for j in range(nch)
    ]
    outs = []
    for j in range(nch):
        gathers[j].wait()
        outs.append(pltpu.async_copy(rows_v.at[j], out_hbm.at[wid, j], osem))
    for o in outs:
        o.wait()


def _sc_gather(table, idx):
    k, d = table.shape
    bn = idx.shape[0]
    b_per_w = bn // _NW
    nch = b_per_w // _CHUNK
    idx3 = idx.reshape(_NW, nch, _CHUNK)
    gathered = pl.kernel(
        _sc_gather_body,
        mesh=plsc.VectorSubcoreMesh(core_axis_name="c", subcore_axis_name="s"),
        out_type=jax.ShapeDtypeStruct((_NW, nch, _CHUNK, d), jnp.float32),
        scratch_types=[
            pltpu.VMEM((nch, _CHUNK), jnp.int32),
            pltpu.VMEM((nch, _CHUNK, d), jnp.float32),
        ] + [pltpu.SemaphoreType.DMA] * (nch + 1),
    )(table, idx3)
    return gathered.reshape(bn, d)


def kernel(x, embed):
    x = x.astype(jnp.float32)
    b, n, d = x.shape
    k = embed.shape[1]
    xf = x.reshape(b * n, d)
    table = embed[0].astype(jnp.float32)

    dist, idx_flat = _dist_argmax(xf, table, row_tile=512)

    quantize = _sc_gather(table, idx_flat).reshape(b, n, d)
    embed_ind = idx_flat.reshape(b, n)
    dist_out = dist.reshape(1, b, n, k)
    return quantize, embed_ind, dist_out


# speedup vs baseline: 3.7881x; 1.0062x over previous
import functools

import jax
import jax.numpy as jnp
from jax import lax
from jax.experimental import pallas as pl
from jax.experimental.pallas import tpu as pltpu
from jax.experimental.pallas import tpu_sc as plsc

_NC, _NS = 2, 16
_NW = _NC * _NS
_CHUNK = 128


def _dist_argmax_body(x_ref, et_ref, dist_ref, idx_ref):
    k = dist_ref.shape[1]
    nchunks = 16
    ck = k // nchunks
    x = x_ref[...]
    chunk_max = []
    chunk_arg = []
    for c in range(nchunks):
        d = lax.dot_general(
            x, et_ref[c * ck:(c + 1) * ck, :],
            dimension_numbers=(((1,), (1,)), ((), ())),
            preferred_element_type=jnp.float32,
        )
        dist_ref[:, c * ck:(c + 1) * ck] = d
        m = jnp.max(d, axis=1, keepdims=True)
        iota = lax.broadcasted_iota(jnp.int32, d.shape, 1).astype(jnp.float32)
        arg = jnp.min(jnp.where(d == m, iota, float(ck)), axis=1)
        chunk_max.append(m[:, 0])
        chunk_arg.append(arg + float(c * ck))
    vals = jnp.stack(chunk_max, axis=1)
    args = jnp.stack(chunk_arg, axis=1)
    gm = jnp.max(vals, axis=1, keepdims=True)
    pick = jnp.min(jnp.where(vals == gm, args, float(k)), axis=1)
    idx_ref[0, 0, :] = pick.astype(jnp.int32)


def _dist_argmax(xf, et, row_tile):
    bn, d = xf.shape
    k = et.shape[0]
    nt = bn // row_tile
    dist, idx3 = pl.pallas_call(
        _dist_argmax_body,
        grid=(nt,),
        in_specs=[
            pl.BlockSpec((row_tile, d), lambda i: (i, 0)),
            pl.BlockSpec((k, d), lambda i: (0, 0)),
        ],
        out_specs=[
            pl.BlockSpec((row_tile, k), lambda i: (i, 0)),
            pl.BlockSpec((1, 1, row_tile), lambda i: (i, 0, 0)),
        ],
        out_shape=[
            jax.ShapeDtypeStruct((bn, k), jnp.float32),
            jax.ShapeDtypeStruct((nt, 1, row_tile), jnp.int32),
        ],
        compiler_params=pltpu.CompilerParams(
            dimension_semantics=("parallel",),
        ),
    )(xf, et)
    return dist, idx3.reshape(bn)


def _sc_gather_body(table_hbm, idx_hbm, out_hbm, idx_v, rows_v, *sems):
    wid = lax.axis_index("s") * _NC + lax.axis_index("c")
    pltpu.sync_copy(idx_hbm.at[wid], idx_v)
    nch = idx_v.shape[0]
    osem = sems[nch]
    gathers = [
        pltpu.async_copy(table_hbm.at[idx_v.at[j]], rows_v.at[j], sems[j])
        for j in range(nch)
    ]
    outs = []
    for j in range(nch):
        gathers[j].wait()
        outs.append(pltpu.async_copy(rows_v.at[j], out_hbm.at[wid, j], osem))
    for o in outs:
        o.wait()


def _sc_gather(table, idx):
    k, d = table.shape
    bn = idx.shape[0]
    b_per_w = bn // _NW
    nch = b_per_w // _CHUNK
    idx3 = idx.reshape(_NW, nch, _CHUNK)
    gathered = pl.kernel(
        _sc_gather_body,
        mesh=plsc.VectorSubcoreMesh(core_axis_name="c", subcore_axis_name="s"),
        out_type=jax.ShapeDtypeStruct((_NW, nch, _CHUNK, d), jnp.float32),
        scratch_types=[
            pltpu.VMEM((nch, _CHUNK), jnp.int32),
            pltpu.VMEM((nch, _CHUNK, d), jnp.float32),
        ] + [pltpu.SemaphoreType.DMA] * (nch + 1),
    )(table, idx3)
    return gathered.reshape(bn, d)


def kernel(x, embed):
    x = x.astype(jnp.float32)
    b, n, d = x.shape
    k = embed.shape[1]
    xf = x.reshape(b * n, d)
    table = embed[0].astype(jnp.float32)

    dist, idx_flat = _dist_argmax(xf, table, row_tile=512)

    quantize = _sc_gather(table, idx_flat).reshape(b, n, d)
    embed_ind = idx_flat.reshape(b, n)
    dist_out = dist.reshape(1, b, n, k)
    return quantize, embed_ind, dist_out
